# baseline (speedup 1.0000x reference)
import functools

import jax
import jax.numpy as jnp
from jax import lax
from jax.experimental import pallas as pl
from jax.experimental.pallas import tpu as pltpu
from jax.experimental.pallas import tpu_sc as plsc

SEQ_LEN = 8192
D_MODEL = 768
NC = 2
NS = 16
NW = NC * NS
ROWS_PER_W = SEQ_LEN // NW

_mesh = plsc.VectorSubcoreMesh(core_axis_name="c", subcore_axis_name="s")


@functools.partial(
    pl.kernel,
    mesh=_mesh,
    out_type=jax.ShapeDtypeStruct((SEQ_LEN, D_MODEL), jnp.float32),
)
def _sc_lookup(table_hbm, out_hbm):
    wid = lax.axis_index("s") * NC + lax.axis_index("c")
    base = wid * ROWS_PER_W
    pltpu.sync_copy(
        table_hbm.at[pl.ds(base, ROWS_PER_W)],
        out_hbm.at[pl.ds(base, ROWS_PER_W)],
    )


def kernel(x, pos_table):
    del x
    return _sc_lookup(pos_table)
